# baseline (speedup 1.0000x reference)
import functools
import jax
import jax.numpy as jnp
from jax import lax
from jax.experimental import pallas as pl
from jax.experimental.pallas import tpu as pltpu
from jax.experimental.pallas import tpu_sc as plsc

_MAX_REL = 32
_S = 512
_D = 768
_NROWS = 2 * _MAX_REL + 1
_NC = 2
_NS = 16
_CBLKS = 16
_RBLKS = 2
_CW = _D // _CBLKS
_RH = _S // _RBLKS
_L = 16
_CV = _CW // _L


def _rpe_sc_body(table_hbm, out_hbm, tbl_v, out_v):
    wid = lax.axis_index("s") * _NC + lax.axis_index("c")
    cb = wid % _CBLKS
    rb = wid // _CBLKS
    c0 = cb * _CW
    j0 = rb * _RH

    pltpu.sync_copy(table_hbm.at[:, pl.ds(c0, _CW)], tbl_v)

    def init_body(r, acc):
        v = r - _MAX_REL
        inner = ((r >= 1) & (r <= _NROWS - 2) & (v <= j0) & (v >= j0 - (_S - 1)))
        w = inner.astype(jnp.float32)
        w = jnp.where(r == 0, jnp.maximum(_S - _MAX_REL - j0, 0).astype(jnp.float32), w)
        w = jnp.where(r == _NROWS - 1, jnp.maximum(j0 - (_MAX_REL - 1), 0).astype(jnp.float32), w)
        return tuple(acc[k] + w * tbl_v[r, pl.ds(k * _L, _L)] for k in range(_CV))

    zero = tuple(jnp.zeros((_L,), jnp.float32) for _ in range(_CV))
    acc = lax.fori_loop(0, _NROWS, init_body, zero)

    inv = jnp.float32(1.0 / _S)
    for k in range(_CV):
        out_v[0, pl.ds(k * _L, _L)] = acc[k] * inv

    def row_body(s, acc):
        j = j0 + s
        hi = jnp.minimum(j, _MAX_REL) + _MAX_REL
        lo = jnp.maximum(j - (_S - _MAX_REL), 0)
        nxt = []
        for k in range(_CV):
            sl = pl.ds(k * _L, _L)
            a = acc[k] + (tbl_v[hi, sl] - tbl_v[lo, sl])
            out_v[s, sl] = a * inv
            nxt.append(a)
        return tuple(nxt)

    lax.fori_loop(1, _RH, row_body, acc)

    pltpu.sync_copy(out_v, out_hbm.at[pl.ds(j0, _RH), pl.ds(c0, _CW)])


def kernel(seq_len, table):
    mesh = plsc.VectorSubcoreMesh(
        core_axis_name="c", subcore_axis_name="s", num_cores=_NC, num_subcores=_NS
    )
    rpe = functools.partial(
        pl.kernel,
        out_type=jax.ShapeDtypeStruct((_S, _D), jnp.float32),
        mesh=mesh,
        scratch_types=[
            pltpu.VMEM((_NROWS, _CW), jnp.float32),
            pltpu.VMEM((_RH, _CW), jnp.float32),
        ],
        compiler_params=pltpu.CompilerParams(use_tc_tiling_on_sc=False),
    )(_rpe_sc_body)
    return rpe(table)[None, :, :]
